# Initial kernel scaffold; baseline (speedup 1.0000x reference)
import functools
import jax
import jax.numpy as jnp
from jax.experimental import pallas as pl
from jax.experimental.pallas import tpu as pltpu

CAP = 100000
D = 256
VDIM = 64
K = 5
BLK = 2000
NB = CAP // BLK

NEG = jnp.float32(-jnp.inf)
IMAX = jnp.int32(2**31 - 1)


def _stream_kernel(query_ref, W1_ref, b1_ref, W2_ref, b2_ref, gamma_ref,
                   beta_ref, keys_ref, tv_ref, ti_ref, qn_ref):
    i = pl.program_id(0)

    @pl.when(i == 0)
    def _():
        q = query_ref[...]
        h = jnp.dot(q, W1_ref[...], preferred_element_type=jnp.float32) + b1_ref[...]
        h = h * jax.nn.sigmoid(h)
        h = jnp.dot(h, W2_ref[...], preferred_element_type=jnp.float32) + b2_ref[...]
        mean = jnp.mean(h, axis=-1, keepdims=True)
        var = jnp.mean((h - mean) * (h - mean), axis=-1, keepdims=True)
        h = (h - mean) * jax.lax.rsqrt(var + 1e-5) * gamma_ref[...] + beta_ref[...]
        n = jnp.sqrt(jnp.sum(h * h))
        qn_ref[...] = h / jnp.maximum(n, 1e-12)

    x = keys_ref[...]
    qn = qn_ref[...]
    dot = jax.lax.dot_general(x, qn, (((1,), (1,)), ((), ())),
                              preferred_element_type=jnp.float32)
    ss = jnp.sum(x * x, axis=1, keepdims=True)
    sim = dot / jnp.maximum(jnp.sqrt(ss), 1e-12)

    iota = jax.lax.broadcasted_iota(jnp.int32, (BLK, 1), 0) + i * BLK
    lane = jax.lax.broadcasted_iota(jnp.int32, (1, 1, 128), 2)
    vals_vec = jnp.full((1, 1, 128), NEG, dtype=jnp.float32)
    idx_vec = jnp.zeros((1, 1, 128), dtype=jnp.int32)
    for j in range(K):
        m = jnp.max(sim)
        sel = jnp.min(jnp.where(sim == m, iota, IMAX))
        vals_vec = jnp.where(lane == j, m, vals_vec)
        idx_vec = jnp.where(lane == j, sel, idx_vec)
        sim = jnp.where(iota == sel, NEG, sim)
    tv_ref[...] = vals_vec
    ti_ref[...] = idx_vec


def _merge_kernel(tv_ref, ti_ref, w_ref, idx_ref):
    tv = tv_ref[...]
    ti = ti_ref[...]
    pos = (jax.lax.broadcasted_iota(jnp.int32, (NB, 1, 128), 0) * 128
           + jax.lax.broadcasted_iota(jnp.int32, (NB, 1, 128), 2))
    lane = jax.lax.broadcasted_iota(jnp.int32, (1, 128), 1)

    sims = []
    idxs = []
    for j in range(K):
        m = jnp.max(tv)
        p = jnp.min(jnp.where(tv == m, pos, IMAX))
        ridx = jnp.max(jnp.where(pos == p, ti, 0))
        sims.append(m)
        idxs.append(ridx)
        tv = jnp.where(pos == p, NEG, tv)

    m0 = sims[0]
    es = [jnp.exp(s - m0) for s in sims]
    denom = es[0] + es[1] + es[2] + es[3] + es[4]

    w_out = jnp.zeros((1, 128), dtype=jnp.float32)
    i_out = jnp.zeros((1, 128), dtype=jnp.int32)
    for j in range(K):
        w_out = jnp.where(lane == j, es[j] / denom, w_out)
        i_out = jnp.where(lane == j, idxs[j], i_out)
    w_ref[...] = w_out
    idx_ref[...] = i_out


def _gather_kernel(idx_ref, w_ref, values_ref, out_ref):
    i = pl.program_id(0)

    @pl.when(i == 0)
    def _():
        out_ref[...] = jnp.zeros_like(out_ref)

    out_ref[...] += w_ref[i] * values_ref[0]


@jax.jit
def kernel(query, keys, values, W1, b1, W2, b2, gamma, beta):
    b1r = b1.reshape(1, D)
    b2r = b2.reshape(1, D)
    gr = gamma.reshape(1, D)
    br = beta.reshape(1, D)

    tv, ti = pl.pallas_call(
        _stream_kernel,
        grid=(NB,),
        in_specs=[
            pl.BlockSpec((1, D), lambda i: (0, 0)),
            pl.BlockSpec((D, D), lambda i: (0, 0)),
            pl.BlockSpec((1, D), lambda i: (0, 0)),
            pl.BlockSpec((D, D), lambda i: (0, 0)),
            pl.BlockSpec((1, D), lambda i: (0, 0)),
            pl.BlockSpec((1, D), lambda i: (0, 0)),
            pl.BlockSpec((1, D), lambda i: (0, 0)),
            pl.BlockSpec((BLK, D), lambda i: (i, 0)),
        ],
        out_specs=[
            pl.BlockSpec((1, 1, 128), lambda i: (i, 0, 0)),
            pl.BlockSpec((1, 1, 128), lambda i: (i, 0, 0)),
        ],
        out_shape=[
            jax.ShapeDtypeStruct((NB, 1, 128), jnp.float32),
            jax.ShapeDtypeStruct((NB, 1, 128), jnp.int32),
        ],
        scratch_shapes=[pltpu.VMEM((1, D), jnp.float32)],
    )(query, W1, b1r, W2, b2r, gr, br, keys)

    w, idx = pl.pallas_call(
        _merge_kernel,
        out_shape=[
            jax.ShapeDtypeStruct((1, 128), jnp.float32),
            jax.ShapeDtypeStruct((1, 128), jnp.int32),
        ],
    )(tv, ti)

    values3 = values.reshape(CAP, 1, VDIM)
    out = pl.pallas_call(
        _gather_kernel,
        grid_spec=pltpu.PrefetchScalarGridSpec(
            num_scalar_prefetch=2,
            grid=(K,),
            in_specs=[
                pl.BlockSpec((1, 1, VDIM), lambda i, idx_ref, w_ref: (idx_ref[i], 0, 0)),
            ],
            out_specs=pl.BlockSpec((1, VDIM), lambda i, idx_ref, w_ref: (0, 0)),
        ),
        out_shape=jax.ShapeDtypeStruct((1, VDIM), jnp.float32),
    )(idx[0], w[0], values3)

    return out.reshape(VDIM)


# speedup vs baseline: 1.4843x; 1.4843x over previous
import functools
import jax
import jax.numpy as jnp
from jax.experimental import pallas as pl
from jax.experimental.pallas import tpu as pltpu

CAP = 100000
D = 256
VDIM = 64
K = 5
BLK = 2000
NB = CAP // BLK

NEG = float("-inf")
IMAX = 2**31 - 1


def _stream_kernel(query_ref, W1_ref, b1_ref, W2_ref, b2_ref, gamma_ref,
                   beta_ref, keys_ref, tv_ref, ti_ref, qn_ref):
    i = pl.program_id(0)

    @pl.when(i == 0)
    def _():
        q = query_ref[...]
        h = jnp.dot(q, W1_ref[...], preferred_element_type=jnp.float32) + b1_ref[...]
        h = h * jax.nn.sigmoid(h)
        h = jnp.dot(h, W2_ref[...], preferred_element_type=jnp.float32) + b2_ref[...]
        mean = jnp.mean(h, axis=-1, keepdims=True)
        var = jnp.mean((h - mean) * (h - mean), axis=-1, keepdims=True)
        h = (h - mean) * jax.lax.rsqrt(var + 1e-5) * gamma_ref[...] + beta_ref[...]
        n = jnp.sqrt(jnp.sum(h * h))
        qn_ref[...] = h / jnp.maximum(n, 1e-12)

    x = keys_ref[...]
    qn = qn_ref[...]
    dot = jax.lax.dot_general(x, qn, (((1,), (1,)), ((), ())),
                              preferred_element_type=jnp.float32)
    ss = jnp.sum(x * x, axis=1, keepdims=True)
    sim = dot / jnp.maximum(jnp.sqrt(ss), 1e-12)

    iota = jax.lax.broadcasted_iota(jnp.int32, (BLK, 1), 0) + i * BLK
    lane = jax.lax.broadcasted_iota(jnp.int32, (1, 1, 128), 2)
    vals_vec = jnp.full((1, 1, 128), NEG, dtype=jnp.float32)
    idx_vec = jnp.zeros((1, 1, 128), dtype=jnp.int32)
    for j in range(K):
        m = jnp.max(sim)
        sel = jnp.min(jnp.where(sim == m, iota, IMAX))
        vals_vec = jnp.where(lane == j, m, vals_vec)
        idx_vec = jnp.where(lane == j, sel, idx_vec)
        sim = jnp.where(iota == sel, NEG, sim)
    tv_ref[...] = vals_vec
    ti_ref[...] = idx_vec


def _merge_kernel(tv_ref, ti_ref, w_ref, idx_ref):
    tv = tv_ref[...]
    ti = ti_ref[...]
    pos = (jax.lax.broadcasted_iota(jnp.int32, (NB, 1, 128), 0) * 128
           + jax.lax.broadcasted_iota(jnp.int32, (NB, 1, 128), 2))
    lane = jax.lax.broadcasted_iota(jnp.int32, (1, 128), 1)

    sims = []
    idxs = []
    for j in range(K):
        m = jnp.max(tv)
        p = jnp.min(jnp.where(tv == m, pos, IMAX))
        ridx = jnp.max(jnp.where(pos == p, ti, 0))
        sims.append(m)
        idxs.append(ridx)
        tv = jnp.where(pos == p, NEG, tv)

    m0 = sims[0]
    es = [jnp.exp(s - m0) for s in sims]
    denom = es[0] + es[1] + es[2] + es[3] + es[4]

    w_out = jnp.zeros((1, 128), dtype=jnp.float32)
    i_out = jnp.zeros((1, 128), dtype=jnp.int32)
    for j in range(K):
        w_out = jnp.where(lane == j, es[j] / denom, w_out)
        i_out = jnp.where(lane == j, idxs[j], i_out)
    w_ref[...] = w_out
    idx_ref[...] = i_out


def _gather_kernel(idx_ref, w_ref, values_ref, out_ref):
    i = pl.program_id(0)

    @pl.when(i == 0)
    def _():
        out_ref[...] = jnp.zeros_like(out_ref)

    out_ref[...] += w_ref[i] * values_ref[0]


@jax.jit
def kernel(query, keys, values, W1, b1, W2, b2, gamma, beta):
    b1r = b1.reshape(1, D)
    b2r = b2.reshape(1, D)
    gr = gamma.reshape(1, D)
    br = beta.reshape(1, D)

    tv, ti = pl.pallas_call(
        _stream_kernel,
        grid=(NB,),
        in_specs=[
            pl.BlockSpec((1, D), lambda i: (0, 0)),
            pl.BlockSpec((D, D), lambda i: (0, 0)),
            pl.BlockSpec((1, D), lambda i: (0, 0)),
            pl.BlockSpec((D, D), lambda i: (0, 0)),
            pl.BlockSpec((1, D), lambda i: (0, 0)),
            pl.BlockSpec((1, D), lambda i: (0, 0)),
            pl.BlockSpec((1, D), lambda i: (0, 0)),
            pl.BlockSpec((BLK, D), lambda i: (i, 0)),
        ],
        out_specs=[
            pl.BlockSpec((1, 1, 128), lambda i: (i, 0, 0)),
            pl.BlockSpec((1, 1, 128), lambda i: (i, 0, 0)),
        ],
        out_shape=[
            jax.ShapeDtypeStruct((NB, 1, 128), jnp.float32),
            jax.ShapeDtypeStruct((NB, 1, 128), jnp.int32),
        ],
        scratch_shapes=[pltpu.VMEM((1, D), jnp.float32)],
    )(query, W1, b1r, W2, b2r, gr, br, keys)

    w, idx = pl.pallas_call(
        _merge_kernel,
        out_shape=[
            jax.ShapeDtypeStruct((1, 128), jnp.float32),
            jax.ShapeDtypeStruct((1, 128), jnp.int32),
        ],
    )(tv, ti)

    values3 = values.reshape(CAP, 1, VDIM)
    out = pl.pallas_call(
        _gather_kernel,
        grid_spec=pltpu.PrefetchScalarGridSpec(
            num_scalar_prefetch=2,
            grid=(K,),
            in_specs=[
                pl.BlockSpec((1, 1, VDIM), lambda i, idx_ref, w_ref: (idx_ref[i], 0, 0)),
            ],
            out_specs=pl.BlockSpec((1, VDIM), lambda i, idx_ref, w_ref: (0, 0)),
        ),
        out_shape=jax.ShapeDtypeStruct((1, VDIM), jnp.float32),
    )(idx[0], w[0], values3)

    return out.reshape(VDIM)


# speedup vs baseline: 1.8439x; 1.2422x over previous
import functools
import jax
import jax.numpy as jnp
from jax.experimental import pallas as pl
from jax.experimental.pallas import tpu as pltpu

CAP = 100000
D = 256
VDIM = 64
K = 5
BLK = 2048
NB = (CAP + BLK - 1) // BLK
G = BLK // 128

NEG = float("-inf")
IMAX = 2**31 - 1


def _stream_kernel(query_ref, W1_ref, b1_ref, W2_ref, b2_ref, gamma_ref,
                   beta_ref, keys_ref, tv_ref, ti_ref, qn_ref):
    i = pl.program_id(0)

    @pl.when(i == 0)
    def _():
        q = query_ref[...]
        h = jnp.dot(q, W1_ref[...], preferred_element_type=jnp.float32) + b1_ref[...]
        h = h * jax.nn.sigmoid(h)
        h = jnp.dot(h, W2_ref[...], preferred_element_type=jnp.float32) + b2_ref[...]
        mean = jnp.mean(h, axis=-1, keepdims=True)
        var = jnp.mean((h - mean) * (h - mean), axis=-1, keepdims=True)
        h = (h - mean) * jax.lax.rsqrt(var + 1e-5) * gamma_ref[...] + beta_ref[...]
        n = jnp.sqrt(jnp.sum(h * h))
        qn_ref[...] = h / jnp.maximum(n, 1e-12)

    x3 = keys_ref[...].reshape(G, 128, D)
    qn = qn_ref[...].reshape(1, 1, D)
    dot = jnp.sum(x3 * qn, axis=2)
    ss = jnp.sum(x3 * x3, axis=2)
    sim = dot / jnp.maximum(jnp.sqrt(ss), 1e-12)

    iota = (jax.lax.broadcasted_iota(jnp.int32, (G, 128), 0) * 128
            + jax.lax.broadcasted_iota(jnp.int32, (G, 128), 1) + i * BLK)
    sim = jnp.where(iota < CAP, sim, NEG)
    lane = jax.lax.broadcasted_iota(jnp.int32, (1, 1, 128), 2)
    vals_vec = jnp.full((1, 1, 128), NEG, dtype=jnp.float32)
    idx_vec = jnp.zeros((1, 1, 128), dtype=jnp.int32)
    for j in range(K):
        m = jnp.max(sim)
        sel = jnp.min(jnp.where(sim == m, iota, IMAX))
        vals_vec = jnp.where(lane == j, m, vals_vec)
        idx_vec = jnp.where(lane == j, sel, idx_vec)
        sim = jnp.where(iota == sel, NEG, sim)
    tv_ref[...] = vals_vec
    ti_ref[...] = idx_vec


def _merge_kernel(tv_ref, ti_ref, w_ref, idx_ref):
    tv = tv_ref[...]
    ti = ti_ref[...]
    pos = (jax.lax.broadcasted_iota(jnp.int32, (NB, 1, 128), 0) * 128
           + jax.lax.broadcasted_iota(jnp.int32, (NB, 1, 128), 2))
    lane = jax.lax.broadcasted_iota(jnp.int32, (1, 128), 1)

    sims = []
    idxs = []
    for j in range(K):
        m = jnp.max(tv)
        p = jnp.min(jnp.where(tv == m, pos, IMAX))
        ridx = jnp.max(jnp.where(pos == p, ti, 0))
        sims.append(m)
        idxs.append(ridx)
        tv = jnp.where(pos == p, NEG, tv)

    m0 = sims[0]
    es = [jnp.exp(s - m0) for s in sims]
    denom = es[0] + es[1] + es[2] + es[3] + es[4]

    w_out = jnp.zeros((1, 128), dtype=jnp.float32)
    i_out = jnp.zeros((1, 128), dtype=jnp.int32)
    for j in range(K):
        w_out = jnp.where(lane == j, es[j] / denom, w_out)
        i_out = jnp.where(lane == j, idxs[j], i_out)
    w_ref[...] = w_out
    idx_ref[...] = i_out


def _gather_kernel(idx_ref, w_ref, values_ref, out_ref):
    i = pl.program_id(0)

    @pl.when(i == 0)
    def _():
        out_ref[...] = jnp.zeros_like(out_ref)

    out_ref[...] += w_ref[i] * values_ref[0]


@jax.jit
def kernel(query, keys, values, W1, b1, W2, b2, gamma, beta):
    b1r = b1.reshape(1, D)
    b2r = b2.reshape(1, D)
    gr = gamma.reshape(1, D)
    br = beta.reshape(1, D)

    tv, ti = pl.pallas_call(
        _stream_kernel,
        grid=(NB,),
        in_specs=[
            pl.BlockSpec((1, D), lambda i: (0, 0)),
            pl.BlockSpec((D, D), lambda i: (0, 0)),
            pl.BlockSpec((1, D), lambda i: (0, 0)),
            pl.BlockSpec((D, D), lambda i: (0, 0)),
            pl.BlockSpec((1, D), lambda i: (0, 0)),
            pl.BlockSpec((1, D), lambda i: (0, 0)),
            pl.BlockSpec((1, D), lambda i: (0, 0)),
            pl.BlockSpec((BLK, D), lambda i: (i, 0)),
        ],
        out_specs=[
            pl.BlockSpec((1, 1, 128), lambda i: (i, 0, 0)),
            pl.BlockSpec((1, 1, 128), lambda i: (i, 0, 0)),
        ],
        out_shape=[
            jax.ShapeDtypeStruct((NB, 1, 128), jnp.float32),
            jax.ShapeDtypeStruct((NB, 1, 128), jnp.int32),
        ],
        scratch_shapes=[pltpu.VMEM((1, D), jnp.float32)],
    )(query, W1, b1r, W2, b2r, gr, br, keys)

    w, idx = pl.pallas_call(
        _merge_kernel,
        out_shape=[
            jax.ShapeDtypeStruct((1, 128), jnp.float32),
            jax.ShapeDtypeStruct((1, 128), jnp.int32),
        ],
    )(tv, ti)

    values3 = values.reshape(CAP, 1, VDIM)
    out = pl.pallas_call(
        _gather_kernel,
        grid_spec=pltpu.PrefetchScalarGridSpec(
            num_scalar_prefetch=2,
            grid=(K,),
            in_specs=[
                pl.BlockSpec((1, 1, VDIM), lambda i, idx_ref, w_ref: (idx_ref[i], 0, 0)),
            ],
            out_specs=pl.BlockSpec((1, VDIM), lambda i, idx_ref, w_ref: (0, 0)),
        ),
        out_shape=jax.ShapeDtypeStruct((1, VDIM), jnp.float32),
    )(idx[0], w[0], values3)

    return out.reshape(VDIM)


# speedup vs baseline: 2.7920x; 1.5142x over previous
import functools
import jax
import jax.numpy as jnp
from jax.experimental import pallas as pl
from jax.experimental.pallas import tpu as pltpu

CAP = 100000
D = 256
VDIM = 64
K = 5
BLK = 16384
NB = (CAP + BLK - 1) // BLK
G = BLK // 128

NEG = float("-inf")
IMAX = 2**31 - 1


def _stream_kernel(query_ref, W1_ref, b1_ref, W2_ref, b2_ref, gamma_ref,
                   beta_ref, keys_ref, tv_ref, ti_ref, qn_ref):
    i = pl.program_id(0)

    @pl.when(i == 0)
    def _():
        q = query_ref[...]
        h = jnp.dot(q, W1_ref[...], preferred_element_type=jnp.float32) + b1_ref[...]
        h = h * jax.nn.sigmoid(h)
        h = jnp.dot(h, W2_ref[...], preferred_element_type=jnp.float32) + b2_ref[...]
        mean = jnp.mean(h, axis=-1, keepdims=True)
        var = jnp.mean((h - mean) * (h - mean), axis=-1, keepdims=True)
        h = (h - mean) * jax.lax.rsqrt(var + 1e-5) * gamma_ref[...] + beta_ref[...]
        n = jnp.sqrt(jnp.sum(h * h))
        qn_ref[...] = h / jnp.maximum(n, 1e-12)

    x3 = keys_ref[...].reshape(G, 128, D)
    qn = qn_ref[...].reshape(1, 1, D)
    dot = jnp.sum(x3 * qn, axis=2)
    ss = jnp.sum(x3 * x3, axis=2)
    sim = dot / jnp.maximum(jnp.sqrt(ss), 1e-12)

    iota = (jax.lax.broadcasted_iota(jnp.int32, (G, 128), 0) * 128
            + jax.lax.broadcasted_iota(jnp.int32, (G, 128), 1) + i * BLK)
    sim = jnp.where(iota < CAP, sim, NEG)
    lane = jax.lax.broadcasted_iota(jnp.int32, (1, 1, 128), 2)
    vals_vec = jnp.full((1, 1, 128), NEG, dtype=jnp.float32)
    idx_vec = jnp.zeros((1, 1, 128), dtype=jnp.int32)
    for j in range(K):
        m = jnp.max(sim)
        sel = jnp.min(jnp.where(sim == m, iota, IMAX))
        vals_vec = jnp.where(lane == j, m, vals_vec)
        idx_vec = jnp.where(lane == j, sel, idx_vec)
        sim = jnp.where(iota == sel, NEG, sim)
    tv_ref[...] = vals_vec
    ti_ref[...] = idx_vec


def _merge_kernel(tv_ref, ti_ref, w_ref, idx_ref):
    tv = tv_ref[...]
    ti = ti_ref[...]
    pos = (jax.lax.broadcasted_iota(jnp.int32, (NB, 1, 128), 0) * 128
           + jax.lax.broadcasted_iota(jnp.int32, (NB, 1, 128), 2))
    lane = jax.lax.broadcasted_iota(jnp.int32, (1, 128), 1)

    sims = []
    idxs = []
    for j in range(K):
        m = jnp.max(tv)
        p = jnp.min(jnp.where(tv == m, pos, IMAX))
        ridx = jnp.max(jnp.where(pos == p, ti, 0))
        sims.append(m)
        idxs.append(ridx)
        tv = jnp.where(pos == p, NEG, tv)

    m0 = sims[0]
    es = [jnp.exp(s - m0) for s in sims]
    denom = es[0] + es[1] + es[2] + es[3] + es[4]

    w_out = jnp.zeros((1, 128), dtype=jnp.float32)
    i_out = jnp.zeros((1, 128), dtype=jnp.int32)
    for j in range(K):
        w_out = jnp.where(lane == j, es[j] / denom, w_out)
        i_out = jnp.where(lane == j, idxs[j], i_out)
    w_ref[...] = w_out
    idx_ref[...] = i_out


def _gather_kernel(idx_ref, w_ref, values_ref, out_ref):
    i = pl.program_id(0)

    @pl.when(i == 0)
    def _():
        out_ref[...] = jnp.zeros_like(out_ref)

    out_ref[...] += w_ref[i] * values_ref[0]


@jax.jit
def kernel(query, keys, values, W1, b1, W2, b2, gamma, beta):
    b1r = b1.reshape(1, D)
    b2r = b2.reshape(1, D)
    gr = gamma.reshape(1, D)
    br = beta.reshape(1, D)

    tv, ti = pl.pallas_call(
        _stream_kernel,
        grid=(NB,),
        in_specs=[
            pl.BlockSpec((1, D), lambda i: (0, 0)),
            pl.BlockSpec((D, D), lambda i: (0, 0)),
            pl.BlockSpec((1, D), lambda i: (0, 0)),
            pl.BlockSpec((D, D), lambda i: (0, 0)),
            pl.BlockSpec((1, D), lambda i: (0, 0)),
            pl.BlockSpec((1, D), lambda i: (0, 0)),
            pl.BlockSpec((1, D), lambda i: (0, 0)),
            pl.BlockSpec((BLK, D), lambda i: (i, 0)),
        ],
        out_specs=[
            pl.BlockSpec((1, 1, 128), lambda i: (i, 0, 0)),
            pl.BlockSpec((1, 1, 128), lambda i: (i, 0, 0)),
        ],
        out_shape=[
            jax.ShapeDtypeStruct((NB, 1, 128), jnp.float32),
            jax.ShapeDtypeStruct((NB, 1, 128), jnp.int32),
        ],
        scratch_shapes=[pltpu.VMEM((1, D), jnp.float32)],
    )(query, W1, b1r, W2, b2r, gr, br, keys)

    w, idx = pl.pallas_call(
        _merge_kernel,
        out_shape=[
            jax.ShapeDtypeStruct((1, 128), jnp.float32),
            jax.ShapeDtypeStruct((1, 128), jnp.int32),
        ],
    )(tv, ti)

    values3 = values.reshape(CAP, 1, VDIM)
    out = pl.pallas_call(
        _gather_kernel,
        grid_spec=pltpu.PrefetchScalarGridSpec(
            num_scalar_prefetch=2,
            grid=(K,),
            in_specs=[
                pl.BlockSpec((1, 1, VDIM), lambda i, idx_ref, w_ref: (idx_ref[i], 0, 0)),
            ],
            out_specs=pl.BlockSpec((1, VDIM), lambda i, idx_ref, w_ref: (0, 0)),
        ),
        out_shape=jax.ShapeDtypeStruct((1, VDIM), jnp.float32),
    )(idx[0], w[0], values3)

    return out.reshape(VDIM)
